# baseline (speedup 1.0000x reference)
import jax
import jax.numpy as jnp
import numpy as np
from jax import lax
from jax.experimental import pallas as pl
from jax.experimental.pallas import tpu as pltpu
from jax.experimental.pallas import tpu_sc as plsc

NUM_ATOMS = 4608
NUM_IN_DEG = 512
NUM_OUT_DEG = 512
H = 768
HW = H // 2
B = 256
N = 128
F = 9
IPN = F + 2
NC = 2
NS = 16
NW = NC * NS
GPW = B // NW
C = 4
KPG = N // C
IPC = C * IPN
NBLK = H // 32
NBUF = 4

_MASK_HI = np.int32(-65536)


def _sum_chunk(buf, ost, ev2):
    def blk(k, carry):
        w0 = 16 * k
        for i in range(C):
            rows = [i * F + j for j in range(F)] + [F * C + i, (F + 1) * C + i]
            v = buf[rows[0], pl.ds(w0, 16)]
            hi = plsc.bitcast(v & _MASK_HI, jnp.float32)
            lo = plsc.bitcast(v << 16, jnp.float32)
            for r in rows[1:]:
                v = buf[r, pl.ds(w0, 16)]
                hi = hi + plsc.bitcast(v & _MASK_HI, jnp.float32)
                lo = lo + plsc.bitcast(v << 16, jnp.float32)
            base = ev2 + (i * H + 32 * k)
            plsc.store_scatter(ost, [base], lo)
            plsc.store_scatter(ost, [base + 1], hi)
        return carry
    lax.fori_loop(0, NBLK, blk, 0, unroll=False)


def _graph_node_feature_kernel(table_hbm, xa_hbm, xi_hbm, xo_hbm,
                               tok_hbm, out_hbm,
                               idxa_v, idxi_v, idxo_v, bufs, osts, tok_v,
                               sg, ss):
    wid = lax.axis_index("s") * NC + lax.axis_index("c")
    ev2 = jax.lax.iota(jnp.int32, 16) * 2

    pltpu.sync_copy(tok_hbm, tok_v)

    def fire(k, b):
        pltpu.async_copy(table_hbm.at[idxa_v.at[k]],
                         bufs[b].at[pl.ds(0, F * C)], sg[b])
        pltpu.async_copy(table_hbm.at[idxi_v.at[k]],
                         bufs[b].at[pl.ds(F * C, C)], sg[b])
        pltpu.async_copy(table_hbm.at[idxo_v.at[k]],
                         bufs[b].at[pl.ds(F * C + C, C)], sg[b])

    def drain(b):
        pltpu.make_async_copy(table_hbm.at[idxa_v.at[0]],
                              bufs[b].at[pl.ds(0, F * C)], sg[b]).wait()
        pltpu.make_async_copy(table_hbm.at[idxi_v.at[0]],
                              bufs[b].at[pl.ds(F * C, C)], sg[b]).wait()
        pltpu.make_async_copy(table_hbm.at[idxo_v.at[0]],
                              bufs[b].at[pl.ds(F * C + C, C)], sg[b]).wait()

    def graph_body(g, carry):
        gid = wid * GPW + g
        pltpu.sync_copy(xa_hbm.at[gid], idxa_v)
        pltpu.sync_copy(xi_hbm.at[gid], idxi_v)
        pltpu.sync_copy(xo_hbm.at[gid], idxo_v)
        pltpu.sync_copy(tok_v, out_hbm.at[pl.ds(gid * (N + 1) * H, H)])

        for b in range(NBUF - 1):
            fire(b, b)

        def ring(t, c2):
            for b in range(NBUF):
                k = NBUF * t + b

                @pl.when(k + NBUF - 1 < KPG)
                def _fire():
                    fire(k + NBUF - 1, (b + NBUF - 1) % NBUF)

                drain(b)

                @pl.when(t > 0)
                def _wait_prev_store():
                    pltpu.make_async_copy(
                        osts[b], out_hbm.at[pl.ds(0, C * H)], ss[b]).wait()

                _sum_chunk(bufs[b], osts[b], ev2)
                row0 = (gid * (N + 1) + 1 + C * k) * H
                pltpu.async_copy(osts[b], out_hbm.at[pl.ds(row0, C * H)], ss[b])
            return c2

        lax.fori_loop(0, KPG // NBUF, ring, 0, unroll=False)

        for b in range(NBUF):
            pltpu.make_async_copy(
                osts[b], out_hbm.at[pl.ds(0, C * H)], ss[b]).wait()
        return carry

    lax.fori_loop(0, GPW, graph_body, 0, unroll=False)


@jax.jit
def _run(table, xa, xi, xo, tok):
    mesh = plsc.VectorSubcoreMesh(core_axis_name="c", subcore_axis_name="s")
    return pl.kernel(
        _graph_node_feature_kernel,
        out_type=jax.ShapeDtypeStruct((B * (N + 1) * H,), jnp.float32),
        mesh=mesh,
        scratch_types=[
            pltpu.VMEM((KPG, F * C), jnp.int32),
            pltpu.VMEM((KPG, C), jnp.int32),
            pltpu.VMEM((KPG, C), jnp.int32),
            [pltpu.VMEM((IPC, HW), jnp.int32) for _ in range(NBUF)],
            [pltpu.VMEM((C * H,), jnp.float32) for _ in range(NBUF)],
            pltpu.VMEM((H,), jnp.float32),
            [pltpu.SemaphoreType.DMA for _ in range(NBUF)],
            [pltpu.SemaphoreType.DMA for _ in range(NBUF)],
        ],
        compiler_params=pltpu.CompilerParams(
            use_tc_tiling_on_sc=False, needs_layout_passes=False),
    )(table, xa, xi, xo, tok)


def kernel(x, in_degree, out_degree, atom_table, in_table, out_table, graph_token):
    xa = x.astype(jnp.int32).reshape(B, KPG, F * C)
    xi = (in_degree.astype(jnp.int32) + (NUM_ATOMS + 1)).reshape(B, KPG, C)
    xo = (out_degree.astype(jnp.int32)
          + (NUM_ATOMS + 1 + NUM_IN_DEG)).reshape(B, KPG, C)
    table = jnp.concatenate([atom_table, in_table, out_table], axis=0)
    packed = lax.bitcast_convert_type(
        table.astype(jnp.bfloat16).reshape(-1, HW, 2), jnp.int32)
    out = _run(packed, xa, xi, xo, graph_token.reshape(H))
    return out.reshape(B, N + 1, H)


# speedup vs baseline: 1.1343x; 1.1343x over previous
import jax
import jax.numpy as jnp
from jax import lax
from jax.experimental import pallas as pl
from jax.experimental.pallas import tpu as pltpu
from jax.experimental.pallas import tpu_sc as plsc

H = 768
B = 256
N = 128
F = 9
IPN = F + 2
NC = 2
NS = 16
NW = NC * NS
GPW = B // NW
C = 4
KPG = N // C
IPC = C * IPN
LANES = H // 16
NBUF = 2


def _sum_chunk(buf, ost):
    def col(v, carry):
        base = v * 16
        for i in range(C):
            rows = [i * F + j for j in range(F)] + [F * C + i, (F + 1) * C + i]
            acc = buf[rows[0], pl.ds(base, 16)]
            for r in rows[1:]:
                acc = acc + buf[r, pl.ds(base, 16)]
            ost[pl.ds(i * H + base, 16)] = acc
        return carry
    lax.fori_loop(0, LANES, col, 0, unroll=False)


def _graph_node_feature_kernel(at_hbm, it_hbm, ot_hbm, xa_hbm, xi_hbm, xo_hbm,
                               tok_hbm, out_hbm,
                               idxa_v, idxi_v, idxo_v, bufs, osts, tok_v,
                               sg, ss):
    wid = lax.axis_index("s") * NC + lax.axis_index("c")

    pltpu.sync_copy(tok_hbm, tok_v)

    def fire(k, b):
        pltpu.async_copy(at_hbm.at[idxa_v.at[k]],
                         bufs[b].at[pl.ds(0, F * C)], sg[b])
        pltpu.async_copy(it_hbm.at[idxi_v.at[k]],
                         bufs[b].at[pl.ds(F * C, C)], sg[b])
        pltpu.async_copy(ot_hbm.at[idxo_v.at[k]],
                         bufs[b].at[pl.ds(F * C + C, C)], sg[b])

    def drain(b):
        pltpu.make_async_copy(at_hbm.at[idxa_v.at[0]],
                              bufs[b].at[pl.ds(0, F * C)], sg[b]).wait()
        pltpu.make_async_copy(it_hbm.at[idxi_v.at[0]],
                              bufs[b].at[pl.ds(F * C, C)], sg[b]).wait()
        pltpu.make_async_copy(ot_hbm.at[idxo_v.at[0]],
                              bufs[b].at[pl.ds(F * C + C, C)], sg[b]).wait()

    def graph_body(g, carry):
        gid = wid * GPW + g
        pltpu.sync_copy(xa_hbm.at[gid], idxa_v)
        pltpu.sync_copy(xi_hbm.at[gid], idxi_v)
        pltpu.sync_copy(xo_hbm.at[gid], idxo_v)
        pltpu.sync_copy(tok_v, out_hbm.at[pl.ds(gid * (N + 1) * H, H)])

        for b in range(NBUF - 1):
            fire(b, b)

        def ring(t, c2):
            for b in range(NBUF):
                k = NBUF * t + b

                @pl.when(k + NBUF - 1 < KPG)
                def _fire():
                    fire(k + NBUF - 1, (b + NBUF - 1) % NBUF)

                drain(b)

                @pl.when(t > 0)
                def _wait_prev_store():
                    pltpu.make_async_copy(
                        osts[b], out_hbm.at[pl.ds(0, C * H)], ss[b]).wait()

                _sum_chunk(bufs[b], osts[b])
                row0 = (gid * (N + 1) + 1 + C * k) * H
                pltpu.async_copy(osts[b], out_hbm.at[pl.ds(row0, C * H)], ss[b])
            return c2

        lax.fori_loop(0, KPG // NBUF, ring, 0, unroll=False)

        for b in range(NBUF):
            pltpu.make_async_copy(
                osts[b], out_hbm.at[pl.ds(0, C * H)], ss[b]).wait()
        return carry

    lax.fori_loop(0, GPW, graph_body, 0, unroll=False)


@jax.jit
def _run(at, it, ot, xa, xi, xo, tok):
    mesh = plsc.VectorSubcoreMesh(core_axis_name="c", subcore_axis_name="s")
    return pl.kernel(
        _graph_node_feature_kernel,
        out_type=jax.ShapeDtypeStruct((B * (N + 1) * H,), jnp.float32),
        mesh=mesh,
        scratch_types=[
            pltpu.VMEM((KPG, F * C), jnp.int32),
            pltpu.VMEM((KPG, C), jnp.int32),
            pltpu.VMEM((KPG, C), jnp.int32),
            [pltpu.VMEM((IPC, H), jnp.float32) for _ in range(NBUF)],
            [pltpu.VMEM((C * H,), jnp.float32) for _ in range(NBUF)],
            pltpu.VMEM((H,), jnp.float32),
            [pltpu.SemaphoreType.DMA for _ in range(NBUF)],
            [pltpu.SemaphoreType.DMA for _ in range(NBUF)],
        ],
        compiler_params=pltpu.CompilerParams(
            use_tc_tiling_on_sc=False, needs_layout_passes=False),
    )(at, it, ot, xa, xi, xo, tok)


def kernel(x, in_degree, out_degree, atom_table, in_table, out_table, graph_token):
    xa = x.astype(jnp.int32).reshape(B, KPG, F * C)
    xi = in_degree.astype(jnp.int32).reshape(B, KPG, C)
    xo = out_degree.astype(jnp.int32).reshape(B, KPG, C)
    z = graph_token[0, 0] * jnp.float32(0.0)
    out = _run(atom_table + z, in_table + z, out_table + z, xa, xi, xo,
               graph_token.reshape(H))
    return out.reshape(B, N + 1, H)


# speedup vs baseline: 1.1767x; 1.0374x over previous
import jax
import jax.numpy as jnp
from jax import lax
from jax.experimental import pallas as pl
from jax.experimental.pallas import tpu as pltpu
from jax.experimental.pallas import tpu_sc as plsc

H = 768
B = 256
N = 128
F = 9
IPN = F + 2
NC = 2
NS = 16
NW = NC * NS
GPW = B // NW
C = 4
KPG = N // C
IPC = C * IPN
LANES = H // 16
NBUF = 2


def _sum_chunk(buf, ost):
    def col(v, carry):
        base = v * 16
        for i in range(C):
            rows = [i * F + j for j in range(F)] + [F * C + i, (F + 1) * C + i]
            acc = buf[rows[0], pl.ds(base, 16)]
            for r in rows[1:]:
                acc = acc + buf[r, pl.ds(base, 16)]
            ost[pl.ds(i * H + base, 16)] = acc
        return carry
    lax.fori_loop(0, LANES, col, 0, unroll=False)


def _graph_node_feature_kernel(at_hbm, it_hbm, ot_hbm, xa_hbm, xi_hbm, xo_hbm,
                               tok_hbm, out_hbm,
                               idxa_v, idxi_v, idxo_v, bufs, osts, tok_v,
                               sg, ss):
    wid = lax.axis_index("s") * NC + lax.axis_index("c")

    pltpu.sync_copy(tok_hbm, tok_v)
    pltpu.sync_copy(xa_hbm.at[pl.ds(wid * GPW, GPW)], idxa_v)
    pltpu.sync_copy(xi_hbm.at[pl.ds(wid * GPW, GPW)], idxi_v)
    pltpu.sync_copy(xo_hbm.at[pl.ds(wid * GPW, GPW)], idxo_v)

    def graph_body(g, carry):
        gid = wid * GPW + g

        def fire(k, b):
            pltpu.async_copy(at_hbm.at[idxa_v.at[g].at[k]],
                             bufs[b].at[pl.ds(0, F * C)], sg[b])
            pltpu.async_copy(it_hbm.at[idxi_v.at[g].at[k]],
                             bufs[b].at[pl.ds(F * C, C)], sg[b])
            pltpu.async_copy(ot_hbm.at[idxo_v.at[g].at[k]],
                             bufs[b].at[pl.ds(F * C + C, C)], sg[b])

        def drain(b):
            pltpu.make_async_copy(at_hbm.at[idxa_v.at[0].at[0]],
                                  bufs[b].at[pl.ds(0, F * C)], sg[b]).wait()
            pltpu.make_async_copy(it_hbm.at[idxi_v.at[0].at[0]],
                                  bufs[b].at[pl.ds(F * C, C)], sg[b]).wait()
            pltpu.make_async_copy(ot_hbm.at[idxo_v.at[0].at[0]],
                                  bufs[b].at[pl.ds(F * C + C, C)], sg[b]).wait()

        pltpu.sync_copy(tok_v, out_hbm.at[pl.ds(gid * (N + 1) * H, H)])

        for b in range(NBUF - 1):
            fire(b, b)

        def ring(t, c2):
            for b in range(NBUF):
                k = NBUF * t + b

                @pl.when(k + NBUF - 1 < KPG)
                def _fire():
                    fire(k + NBUF - 1, (b + NBUF - 1) % NBUF)

                drain(b)

                @pl.when(t > 0)
                def _wait_prev_store():
                    pltpu.make_async_copy(
                        osts[b], out_hbm.at[pl.ds(0, C * H)], ss[b]).wait()

                _sum_chunk(bufs[b], osts[b])
                row0 = (gid * (N + 1) + 1 + C * k) * H
                pltpu.async_copy(osts[b], out_hbm.at[pl.ds(row0, C * H)], ss[b])
            return c2

        lax.fori_loop(0, KPG // NBUF, ring, 0, unroll=False)

        for b in range(NBUF):
            pltpu.make_async_copy(
                osts[b], out_hbm.at[pl.ds(0, C * H)], ss[b]).wait()
        return carry

    lax.fori_loop(0, GPW, graph_body, 0, unroll=False)


@jax.jit
def _run(at, it, ot, xa, xi, xo, tok):
    mesh = plsc.VectorSubcoreMesh(core_axis_name="c", subcore_axis_name="s")
    return pl.kernel(
        _graph_node_feature_kernel,
        out_type=jax.ShapeDtypeStruct((B * (N + 1) * H,), jnp.float32),
        mesh=mesh,
        scratch_types=[
            pltpu.VMEM((GPW, KPG, F * C), jnp.int32),
            pltpu.VMEM((GPW, KPG, C), jnp.int32),
            pltpu.VMEM((GPW, KPG, C), jnp.int32),
            [pltpu.VMEM((IPC, H), jnp.float32) for _ in range(NBUF)],
            [pltpu.VMEM((C * H,), jnp.float32) for _ in range(NBUF)],
            pltpu.VMEM((H,), jnp.float32),
            [pltpu.SemaphoreType.DMA for _ in range(NBUF)],
            [pltpu.SemaphoreType.DMA for _ in range(NBUF)],
        ],
        compiler_params=pltpu.CompilerParams(
            use_tc_tiling_on_sc=False, needs_layout_passes=False),
    )(at, it, ot, xa, xi, xo, tok)


def kernel(x, in_degree, out_degree, atom_table, in_table, out_table, graph_token):
    xa = x.astype(jnp.int32).reshape(B, KPG, F * C)
    xi = in_degree.astype(jnp.int32).reshape(B, KPG, C)
    xo = out_degree.astype(jnp.int32).reshape(B, KPG, C)
    out = _run(atom_table, in_table, out_table, xa, xi, xo,
               graph_token.reshape(H))
    return out.reshape(B, N + 1, H)


# speedup vs baseline: 1.2108x; 1.0290x over previous
import jax
import jax.numpy as jnp
from jax import lax
from jax.experimental import pallas as pl
from jax.experimental.pallas import tpu as pltpu
from jax.experimental.pallas import tpu_sc as plsc

H = 768
B = 256
N = 128
F = 9
IPN = F + 2
NC = 2
NS = 16
NW = NC * NS
GPW = B // NW
C = 4
KPG = N // C
IPC = C * IPN
LANES = H // 16
NBUF = 2


def _sum_chunk(buf, ost):
    def col(v, carry):
        base = v * 16
        for i in range(C):
            rows = [i * F + j for j in range(F)] + [F * C + i, (F + 1) * C + i]
            acc = buf[rows[0], pl.ds(base, 16)]
            for r in rows[1:]:
                acc = acc + buf[r, pl.ds(base, 16)]
            ost[pl.ds(i * H + base, 16)] = acc
        return carry
    lax.fori_loop(0, LANES, col, 0, unroll=False)


def _graph_node_feature_kernel(at_hbm, it_hbm, ot_hbm, xa_hbm, xi_hbm, xo_hbm,
                               tok_hbm, out_hbm,
                               idxa_v, idxi_v, idxo_v, bufs, osts, tok_v,
                               sg, ss):
    wid = lax.axis_index("s") * NC + lax.axis_index("c")

    pltpu.sync_copy(tok_hbm, tok_v)
    pltpu.sync_copy(xa_hbm.at[pl.ds(wid * GPW, GPW)], idxa_v)
    pltpu.sync_copy(xi_hbm.at[pl.ds(wid * GPW, GPW)], idxi_v)
    pltpu.sync_copy(xo_hbm.at[pl.ds(wid * GPW, GPW)], idxo_v)

    def tok_body(g, carry):
        gid = wid * GPW + g
        pltpu.sync_copy(tok_v, out_hbm.at[pl.ds(gid * (N + 1) * H, H)])
        return carry

    lax.fori_loop(0, GPW, tok_body, 0, unroll=False)

    def fire(q, b):
        g = q // KPG
        k = q % KPG
        pltpu.async_copy(at_hbm.at[idxa_v.at[g].at[k]],
                         bufs[b].at[pl.ds(0, F * C)], sg[b])
        pltpu.async_copy(it_hbm.at[idxi_v.at[g].at[k]],
                         bufs[b].at[pl.ds(F * C, C)], sg[b])
        pltpu.async_copy(ot_hbm.at[idxo_v.at[g].at[k]],
                         bufs[b].at[pl.ds(F * C + C, C)], sg[b])

    def drain(b):
        pltpu.make_async_copy(at_hbm.at[idxa_v.at[0].at[0]],
                              bufs[b].at[pl.ds(0, F * C)], sg[b]).wait()
        pltpu.make_async_copy(it_hbm.at[idxi_v.at[0].at[0]],
                              bufs[b].at[pl.ds(F * C, C)], sg[b]).wait()
        pltpu.make_async_copy(ot_hbm.at[idxo_v.at[0].at[0]],
                              bufs[b].at[pl.ds(F * C + C, C)], sg[b]).wait()

    TCH = GPW * KPG

    for b in range(NBUF - 1):
        fire(b, b)

    def ring(t, c2):
        for b in range(NBUF):
            q = NBUF * t + b

            @pl.when(q + NBUF - 1 < TCH)
            def _fire():
                fire(q + NBUF - 1, (b + NBUF - 1) % NBUF)

            drain(b)

            @pl.when(t > 0)
            def _wait_prev_store():
                pltpu.make_async_copy(
                    osts[b], out_hbm.at[pl.ds(0, C * H)], ss[b]).wait()

            _sum_chunk(bufs[b], osts[b])
            gid = wid * GPW + q // KPG
            row0 = (gid * (N + 1) + 1 + C * (q % KPG)) * H
            pltpu.async_copy(osts[b], out_hbm.at[pl.ds(row0, C * H)], ss[b])
        return c2

    lax.fori_loop(0, TCH // NBUF, ring, 0, unroll=False)

    for b in range(NBUF):
        pltpu.make_async_copy(
            osts[b], out_hbm.at[pl.ds(0, C * H)], ss[b]).wait()


@jax.jit
def _run(at, it, ot, xa, xi, xo, tok):
    mesh = plsc.VectorSubcoreMesh(core_axis_name="c", subcore_axis_name="s")
    return pl.kernel(
        _graph_node_feature_kernel,
        out_type=jax.ShapeDtypeStruct((B * (N + 1) * H,), jnp.float32),
        mesh=mesh,
        scratch_types=[
            pltpu.VMEM((GPW, KPG, F * C), jnp.int32),
            pltpu.VMEM((GPW, KPG, C), jnp.int32),
            pltpu.VMEM((GPW, KPG, C), jnp.int32),
            [pltpu.VMEM((IPC, H), jnp.float32) for _ in range(NBUF)],
            [pltpu.VMEM((C * H,), jnp.float32) for _ in range(NBUF)],
            pltpu.VMEM((H,), jnp.float32),
            [pltpu.SemaphoreType.DMA for _ in range(NBUF)],
            [pltpu.SemaphoreType.DMA for _ in range(NBUF)],
        ],
        compiler_params=pltpu.CompilerParams(
            use_tc_tiling_on_sc=False, needs_layout_passes=False),
    )(at, it, ot, xa, xi, xo, tok)


def kernel(x, in_degree, out_degree, atom_table, in_table, out_table, graph_token):
    xa = x.astype(jnp.int32).reshape(B, KPG, F * C)
    xi = in_degree.astype(jnp.int32).reshape(B, KPG, C)
    xo = out_degree.astype(jnp.int32).reshape(B, KPG, C)
    out = _run(atom_table, in_table, out_table, xa, xi, xo,
               graph_token.reshape(H))
    return out.reshape(B, N + 1, H)
